# baseline (speedup 1.0000x reference)
import jax
import jax.numpy as jnp
from jax import lax
from jax.experimental import pallas as pl
from jax.experimental.pallas import tpu as pltpu

_B, _D = 16384, 2048
_H = 2048
_BR = 256
_G = _B // _BR
_WR = _H // _G
_NBUF = 8


def _loss_kernel(yh_hbm, y_hbm, w1_hbm, w2_hbm, out_ref,
                 yh_b, y_b, w1_b, w2_b, acc_ref,
                 s_yh, s_y, s_w1, s_w2):

    def fetch(s, b):
        pltpu.make_async_copy(
            yh_hbm.at[pl.ds(s * _BR, _BR)], yh_b.at[b], s_yh.at[b]).start()
        pltpu.make_async_copy(
            y_hbm.at[pl.ds(s * _BR, _BR)], y_b.at[b], s_y.at[b]).start()
        pltpu.make_async_copy(
            w1_hbm.at[pl.ds(s * _WR, _WR)], w1_b.at[b], s_w1.at[b]).start()
        pltpu.make_async_copy(
            w2_hbm.at[pl.ds(s * _WR, _WR)], w2_b.at[b], s_w2.at[b]).start()

    for s in range(_NBUF):
        fetch(s, s)

    acc_ref[0] = 0.0
    acc_ref[1] = 0.0
    acc_ref[2] = 0.0

    def step(s, carry):
        b = lax.rem(s, _NBUF)
        pltpu.make_async_copy(
            yh_hbm.at[pl.ds(0, _BR)], yh_b.at[b], s_yh.at[b]).wait()
        pltpu.make_async_copy(
            y_hbm.at[pl.ds(0, _BR)], y_b.at[b], s_y.at[b]).wait()
        pltpu.make_async_copy(
            w1_hbm.at[pl.ds(0, _WR)], w1_b.at[b], s_w1.at[b]).wait()
        pltpu.make_async_copy(
            w2_hbm.at[pl.ds(0, _WR)], w2_b.at[b], s_w2.at[b]).wait()

        d = yh_b[b] - y_b[b]
        acc_ref[0] += jnp.sum(d * d)
        w1 = w1_b[b]
        acc_ref[1] += jnp.sum(w1 * w1)
        w2 = w2_b[b]
        acc_ref[2] += jnp.sum(w2 * w2)

        @pl.when(s + _NBUF < _G)
        def _():
            nxt = s + _NBUF
            pltpu.make_async_copy(
                yh_hbm.at[pl.ds(nxt * _BR, _BR)], yh_b.at[b], s_yh.at[b]).start()
            pltpu.make_async_copy(
                y_hbm.at[pl.ds(nxt * _BR, _BR)], y_b.at[b], s_y.at[b]).start()
            pltpu.make_async_copy(
                w1_hbm.at[pl.ds(nxt * _WR, _WR)], w1_b.at[b], s_w1.at[b]).start()
            pltpu.make_async_copy(
                w2_hbm.at[pl.ds(nxt * _WR, _WR)], w2_b.at[b], s_w2.at[b]).start()
        return carry

    lax.fori_loop(0, _G, step, 0)

    out_ref[0, 0] = (acc_ref[0] / (_B * _D)
                     + jnp.sqrt(acc_ref[1]) + jnp.sqrt(acc_ref[2]))


def kernel(y_hat, y, W1, W2):
    out = pl.pallas_call(
        _loss_kernel,
        in_specs=[
            pl.BlockSpec(memory_space=pl.ANY),
            pl.BlockSpec(memory_space=pl.ANY),
            pl.BlockSpec(memory_space=pl.ANY),
            pl.BlockSpec(memory_space=pl.ANY),
        ],
        out_specs=pl.BlockSpec(memory_space=pltpu.SMEM),
        out_shape=jax.ShapeDtypeStruct((1, 1), jnp.float32),
        scratch_shapes=[
            pltpu.VMEM((_NBUF, _BR, _D), jnp.float32),
            pltpu.VMEM((_NBUF, _BR, _D), jnp.float32),
            pltpu.VMEM((_NBUF, _WR, _D), jnp.float32),
            pltpu.VMEM((_NBUF, _WR, _H), jnp.float32),
            pltpu.SMEM((3,), jnp.float32),
            pltpu.SemaphoreType.DMA((_NBUF,)),
            pltpu.SemaphoreType.DMA((_NBUF,)),
            pltpu.SemaphoreType.DMA((_NBUF,)),
            pltpu.SemaphoreType.DMA((_NBUF,)),
        ],
    )(y_hat, y, W1, W2)
    return out[0, 0]


# speedup vs baseline: 1.0036x; 1.0036x over previous
import jax
import jax.numpy as jnp
from jax import lax
from jax.experimental import pallas as pl
from jax.experimental.pallas import tpu as pltpu

_B, _D = 16384, 2048
_H = 2048
_BR = 128
_G = _B // _BR
_WR = _H // _G
_NBUF = 8


def _loss_kernel(yh_hbm, y_hbm, w1_hbm, w2_hbm, out_ref,
                 yh_b, y_b, w1_b, w2_b, acc_ref,
                 s_yh, s_y, s_w1, s_w2):

    def fetch(s, b):
        pltpu.make_async_copy(
            yh_hbm.at[pl.ds(s * _BR, _BR)], yh_b.at[b], s_yh.at[b]).start()
        pltpu.make_async_copy(
            y_hbm.at[pl.ds(s * _BR, _BR)], y_b.at[b], s_y.at[b]).start()
        pltpu.make_async_copy(
            w1_hbm.at[pl.ds(s * _WR, _WR)], w1_b.at[b], s_w1.at[b]).start()
        pltpu.make_async_copy(
            w2_hbm.at[pl.ds(s * _WR, _WR)], w2_b.at[b], s_w2.at[b]).start()

    for s in range(_NBUF):
        fetch(s, s)

    acc_ref[0] = 0.0
    acc_ref[1] = 0.0
    acc_ref[2] = 0.0

    def step(s, carry):
        b = lax.rem(s, _NBUF)
        pltpu.make_async_copy(
            yh_hbm.at[pl.ds(0, _BR)], yh_b.at[b], s_yh.at[b]).wait()
        pltpu.make_async_copy(
            y_hbm.at[pl.ds(0, _BR)], y_b.at[b], s_y.at[b]).wait()
        pltpu.make_async_copy(
            w1_hbm.at[pl.ds(0, _WR)], w1_b.at[b], s_w1.at[b]).wait()
        pltpu.make_async_copy(
            w2_hbm.at[pl.ds(0, _WR)], w2_b.at[b], s_w2.at[b]).wait()

        d = yh_b[b] - y_b[b]
        acc_ref[0] += jnp.sum(d * d)
        w1 = w1_b[b]
        acc_ref[1] += jnp.sum(w1 * w1)
        w2 = w2_b[b]
        acc_ref[2] += jnp.sum(w2 * w2)

        @pl.when(s + _NBUF < _G)
        def _():
            nxt = s + _NBUF
            pltpu.make_async_copy(
                yh_hbm.at[pl.ds(nxt * _BR, _BR)], yh_b.at[b], s_yh.at[b]).start()
            pltpu.make_async_copy(
                y_hbm.at[pl.ds(nxt * _BR, _BR)], y_b.at[b], s_y.at[b]).start()
            pltpu.make_async_copy(
                w1_hbm.at[pl.ds(nxt * _WR, _WR)], w1_b.at[b], s_w1.at[b]).start()
            pltpu.make_async_copy(
                w2_hbm.at[pl.ds(nxt * _WR, _WR)], w2_b.at[b], s_w2.at[b]).start()
        return carry

    lax.fori_loop(0, _G, step, 0)

    out_ref[0, 0] = (acc_ref[0] / (_B * _D)
                     + jnp.sqrt(acc_ref[1]) + jnp.sqrt(acc_ref[2]))


def kernel(y_hat, y, W1, W2):
    out = pl.pallas_call(
        _loss_kernel,
        in_specs=[
            pl.BlockSpec(memory_space=pl.ANY),
            pl.BlockSpec(memory_space=pl.ANY),
            pl.BlockSpec(memory_space=pl.ANY),
            pl.BlockSpec(memory_space=pl.ANY),
        ],
        out_specs=pl.BlockSpec(memory_space=pltpu.SMEM),
        out_shape=jax.ShapeDtypeStruct((1, 1), jnp.float32),
        scratch_shapes=[
            pltpu.VMEM((_NBUF, _BR, _D), jnp.float32),
            pltpu.VMEM((_NBUF, _BR, _D), jnp.float32),
            pltpu.VMEM((_NBUF, _WR, _D), jnp.float32),
            pltpu.VMEM((_NBUF, _WR, _H), jnp.float32),
            pltpu.SMEM((3,), jnp.float32),
            pltpu.SemaphoreType.DMA((_NBUF,)),
            pltpu.SemaphoreType.DMA((_NBUF,)),
            pltpu.SemaphoreType.DMA((_NBUF,)),
            pltpu.SemaphoreType.DMA((_NBUF,)),
        ],
    )(y_hat, y, W1, W2)
    return out[0, 0]


# speedup vs baseline: 1.0085x; 1.0049x over previous
import jax
import jax.numpy as jnp
from jax import lax
from jax.experimental import pallas as pl
from jax.experimental.pallas import tpu as pltpu

_B, _D = 16384, 2048
_H = 2048
_BR = 256
_G = _B // _BR
_WR = _H // _G
_NBUF = 6


def _loss_kernel(yh_hbm, y_hbm, w1_hbm, w2_hbm, out_ref,
                 yh_b, y_b, w1_b, w2_b, acc_ref,
                 s_yh, s_y, s_w1, s_w2):

    def fetch(s, b):
        pltpu.make_async_copy(
            yh_hbm.at[pl.ds(s * _BR, _BR)], yh_b.at[b], s_yh.at[b]).start()
        pltpu.make_async_copy(
            y_hbm.at[pl.ds(s * _BR, _BR)], y_b.at[b], s_y.at[b]).start()
        pltpu.make_async_copy(
            w1_hbm.at[pl.ds(s * _WR, _WR)], w1_b.at[b], s_w1.at[b]).start()
        pltpu.make_async_copy(
            w2_hbm.at[pl.ds(s * _WR, _WR)], w2_b.at[b], s_w2.at[b]).start()

    for s in range(_NBUF):
        fetch(s, s)

    acc_ref[0] = 0.0
    acc_ref[1] = 0.0
    acc_ref[2] = 0.0

    def step(s, carry):
        b = lax.rem(s, _NBUF)
        pltpu.make_async_copy(
            yh_hbm.at[pl.ds(0, _BR)], yh_b.at[b], s_yh.at[b]).wait()
        pltpu.make_async_copy(
            y_hbm.at[pl.ds(0, _BR)], y_b.at[b], s_y.at[b]).wait()
        pltpu.make_async_copy(
            w1_hbm.at[pl.ds(0, _WR)], w1_b.at[b], s_w1.at[b]).wait()
        pltpu.make_async_copy(
            w2_hbm.at[pl.ds(0, _WR)], w2_b.at[b], s_w2.at[b]).wait()

        d = yh_b[b] - y_b[b]
        acc_ref[0] += jnp.sum(d * d)
        w1 = w1_b[b]
        acc_ref[1] += jnp.sum(w1 * w1)
        w2 = w2_b[b]
        acc_ref[2] += jnp.sum(w2 * w2)

        @pl.when(s + _NBUF < _G)
        def _():
            nxt = s + _NBUF
            pltpu.make_async_copy(
                yh_hbm.at[pl.ds(nxt * _BR, _BR)], yh_b.at[b], s_yh.at[b]).start()
            pltpu.make_async_copy(
                y_hbm.at[pl.ds(nxt * _BR, _BR)], y_b.at[b], s_y.at[b]).start()
            pltpu.make_async_copy(
                w1_hbm.at[pl.ds(nxt * _WR, _WR)], w1_b.at[b], s_w1.at[b]).start()
            pltpu.make_async_copy(
                w2_hbm.at[pl.ds(nxt * _WR, _WR)], w2_b.at[b], s_w2.at[b]).start()
        return carry

    lax.fori_loop(0, _G, step, 0)

    out_ref[0, 0] = (acc_ref[0] / (_B * _D)
                     + jnp.sqrt(acc_ref[1]) + jnp.sqrt(acc_ref[2]))


def kernel(y_hat, y, W1, W2):
    out = pl.pallas_call(
        _loss_kernel,
        in_specs=[
            pl.BlockSpec(memory_space=pl.ANY),
            pl.BlockSpec(memory_space=pl.ANY),
            pl.BlockSpec(memory_space=pl.ANY),
            pl.BlockSpec(memory_space=pl.ANY),
        ],
        out_specs=pl.BlockSpec(memory_space=pltpu.SMEM),
        out_shape=jax.ShapeDtypeStruct((1, 1), jnp.float32),
        scratch_shapes=[
            pltpu.VMEM((_NBUF, _BR, _D), jnp.float32),
            pltpu.VMEM((_NBUF, _BR, _D), jnp.float32),
            pltpu.VMEM((_NBUF, _WR, _D), jnp.float32),
            pltpu.VMEM((_NBUF, _WR, _H), jnp.float32),
            pltpu.SMEM((3,), jnp.float32),
            pltpu.SemaphoreType.DMA((_NBUF,)),
            pltpu.SemaphoreType.DMA((_NBUF,)),
            pltpu.SemaphoreType.DMA((_NBUF,)),
            pltpu.SemaphoreType.DMA((_NBUF,)),
        ],
    )(y_hat, y, W1, W2)
    return out[0, 0]
